# baseline (device time: 133997 ns/iter reference)
import jax
import jax.numpy as jnp
from jax import lax
from jax.experimental import pallas as pl
from jax.experimental.pallas import tpu as pltpu

N_DEV = 4
BN = 1024


def kernel(x, w_mat, scale_x, scale_w):
    m_total, k_shard = x.shape
    k_total, n_total = w_mat.shape
    m_per = m_total // N_DEV
    n_blocks = n_total // BN

    def body(x_ref, w_ref, sx_ref, sw_ref, out_ref,
             x_parts, send_sems, recv_sems):
        my_i = lax.axis_index("i")
        nb = pl.program_id(0)

        @pl.when(nb == 0)
        def _comm():
            barrier_sem = pltpu.get_barrier_semaphore()
            for off in range(1, N_DEV):
                peer = lax.rem(my_i + off, N_DEV)
                pl.semaphore_signal(
                    barrier_sem, inc=1,
                    device_id=(peer,), device_id_type=pl.DeviceIdType.MESH,
                )
            pl.semaphore_wait(barrier_sem, N_DEV - 1)

            sends = []
            for off in range(1, N_DEV):
                peer = lax.rem(my_i + off, N_DEV)
                rdma = pltpu.make_async_remote_copy(
                    src_ref=x_ref.at[pl.ds(peer * m_per, m_per), :],
                    dst_ref=x_parts.at[my_i],
                    send_sem=send_sems.at[off - 1],
                    recv_sem=recv_sems.at[my_i],
                    device_id=(peer,),
                    device_id_type=pl.DeviceIdType.MESH,
                )
                rdma.start()
                sends.append(rdma)

            x_parts[my_i] = x_ref[pl.ds(my_i * m_per, m_per), :]

            for off in range(1, N_DEV):
                j = lax.rem(my_i + off, N_DEV)
                recv = pltpu.make_async_remote_copy(
                    src_ref=x_parts.at[j],
                    dst_ref=x_parts.at[j],
                    send_sem=send_sems.at[off - 1],
                    recv_sem=recv_sems.at[j],
                    device_id=(j,),
                    device_id_type=pl.DeviceIdType.MESH,
                )
                recv.wait_recv()
            for rdma in sends:
                rdma.wait_send()

        acc = jnp.zeros((m_per, BN), jnp.float32)
        for j in range(N_DEV):
            xj = x_parts[j].astype(jnp.bfloat16)
            wj = w_ref[j * k_shard:(j + 1) * k_shard, :].astype(jnp.bfloat16)
            acc = acc + lax.dot_general(
                xj, wj, (((1,), (0,)), ((), ())),
                preferred_element_type=jnp.float32,
            )
        y = acc * (sx_ref[0] * sw_ref[0])
        yc = jnp.clip(y, -60.0, 60.0)
        out_ref[:, :] = y / (1.0 + jnp.exp(-yc))

    return pl.pallas_call(
        body,
        grid=(n_blocks,),
        in_specs=[
            pl.BlockSpec((m_total, k_shard), lambda n: (0, 0)),
            pl.BlockSpec((k_total, BN), lambda n: (0, n)),
            pl.BlockSpec(memory_space=pltpu.SMEM),
            pl.BlockSpec(memory_space=pltpu.SMEM),
        ],
        out_specs=pl.BlockSpec((m_per, BN), lambda n: (0, n)),
        out_shape=jax.ShapeDtypeStruct((m_per, n_total), jnp.float32),
        scratch_shapes=[
            pltpu.VMEM((N_DEV, m_per, k_shard), jnp.int8),
            pltpu.SemaphoreType.DMA((N_DEV - 1,)),
            pltpu.SemaphoreType.DMA((N_DEV,)),
        ],
        compiler_params=pltpu.CompilerParams(
            collective_id=0,
            dimension_semantics=("arbitrary",),
        ),
    )(x, w_mat, scale_x, scale_w)
